# baseline (speedup 1.0000x reference)
import functools

import jax
import jax.numpy as jnp
from jax import lax
from jax.experimental import pallas as pl
from jax.experimental.pallas import tpu as pltpu
from jax.experimental.pallas import tpu_sc as plsc

NC = 2
NS = 16
L = 16
NW = NC * NS
C = 128
K0_NUM, K1_NUM, K_DEN = 28, 52, 80


def _dense_table(h, wfull):
    def body(h_ref, w_ref, o_ref):
        o_ref[...] = jnp.dot(h_ref[...], w_ref[...],
                             preferred_element_type=jnp.float32,
                             precision=jax.lax.Precision.HIGHEST)
    return pl.pallas_call(
        body,
        out_shape=jax.ShapeDtypeStruct((h.shape[0], wfull.shape[1]),
                                       jnp.float32),
    )(h, wfull)


def _combine_relu_table(p0, p1, h, root, bias2d, wfull):
    N = h.shape[0]
    H = root.shape[1]
    def body(p0_ref, p1_ref, h_ref, r_ref, b_ref, w_ref, ho_ref, t_ref):
        hn = p0_ref[...] + p1_ref[...] + jnp.dot(
            h_ref[...], r_ref[...], preferred_element_type=jnp.float32,
                             precision=jax.lax.Precision.HIGHEST)
        hn = jnp.maximum(hn + b_ref[...], 0.0)
        ho_ref[...] = hn
        t_ref[...] = jnp.dot(hn, w_ref[...],
                             preferred_element_type=jnp.float32,
                             precision=jax.lax.Precision.HIGHEST)
    return pl.pallas_call(
        body,
        out_shape=[jax.ShapeDtypeStruct((N, H), jnp.float32),
                   jax.ShapeDtypeStruct((N, wfull.shape[1]), jnp.float32)],
    )(p0, p1, h, root, bias2d, wfull)


def _final_mlp(p0, p1, h, root, bias2d, fc1_W, fc1_b2d, fc2_W, fc2_b2d):
    N = h.shape[0]
    def body(p0_ref, p1_ref, h_ref, r_ref, b_ref, w1_ref, b1_ref,
             w2_ref, b2_ref, o_ref):
        hn = p0_ref[...] + p1_ref[...] + jnp.dot(
            h_ref[...], r_ref[...], preferred_element_type=jnp.float32,
                             precision=jax.lax.Precision.HIGHEST)
        hn = jnp.maximum(hn + b_ref[...], 0.0)
        hn = jnp.maximum(
            jnp.dot(hn, w1_ref[...], preferred_element_type=jnp.float32,
                             precision=jax.lax.Precision.HIGHEST)
            + b1_ref[...], 0.0)
        o_ref[...] = jnp.dot(hn, w2_ref[...],
                             preferred_element_type=jnp.float32,
                             precision=jax.lax.Precision.HIGHEST) + b2_ref[...]
    return pl.pallas_call(
        body,
        out_shape=jax.ShapeDtypeStruct((N, fc2_W.shape[1]), jnp.float32),
    )(p0, p1, h, root, bias2d, fc1_W, fc1_b2d, fc2_W, fc2_b2d)


def _edge_pass(table, ed, aggr, e_pad):
    H = table.shape[1] // 4
    per_pair = e_pad // (NS * C)
    if per_pair >= 8:
        k0 = max(2, (per_pair * K0_NUM // K_DEN) & ~1)
    else:
        k0 = (per_pair // 2) & ~1
    k1 = per_pair - k0
    rows_per_tile = aggr // NS

    mesh = plsc.VectorSubcoreMesh(core_axis_name="c", subcore_axis_name="s",
                                  num_cores=NC, num_subcores=NS)

    @functools.partial(
        pl.kernel,
        out_type=jax.ShapeDtypeStruct((NC, aggr, H), jnp.float32),
        mesh=mesh,
        compiler_params=pltpu.CompilerParams(use_tc_tiling_on_sc=False,
                                             needs_layout_passes=False),
        scratch_types=[
            [pltpu.VMEM((5, C), jnp.int32)] * 2,
            [pltpu.VMEM((C, 4 * H), jnp.float32)] * 2,
            pltpu.VMEM((C, H), jnp.float32),
            pltpu.VMEM_SHARED((aggr, H), jnp.float32),
            [pltpu.SemaphoreType.DMA] * 2,
            [pltpu.SemaphoreType.DMA] * 2,
        ],
    )
    def k(table_h, ed_h, out_h, ed_v, rows_v, msg_v, agg_sh, ed_sem, g_sem):
        c = lax.axis_index("c")
        s = lax.axis_index("s")

        def zbody(e, carry):
            msg_v[e, pl.ds(0, L)] = jnp.zeros((L,), jnp.float32)
            msg_v[e, pl.ds(L, L)] = jnp.zeros((L,), jnp.float32)
            return carry
        lax.fori_loop(0, C, zbody, 0)
        for kb in range(rows_per_tile // C):
            pltpu.sync_copy(msg_v,
                            agg_sh.at[pl.ds(s * rows_per_tile + kb * C, C)])
        plsc.subcore_barrier()

        def combine(b):
            def grp_body(g, gcarry):
                av0 = plsc.bitcast(ed_v[b][2, pl.ds(g * L, L)], jnp.float32)
                av1 = plsc.bitcast(ed_v[b][3, pl.ds(g * L, L)], jnp.float32)
                av2 = plsc.bitcast(ed_v[b][4, pl.ds(g * L, L)], jnp.float32)
                for lane in range(L):
                    e = g * L + lane
                    b0 = av0[lane]
                    b1 = av1[lane]
                    b2 = av2[lane]
                    for j in range(H // L):
                        t0 = rows_v[b][e, pl.ds(j * L, L)]
                        t1 = rows_v[b][e, pl.ds(H + j * L, L)]
                        t2 = rows_v[b][e, pl.ds(2 * H + j * L, L)]
                        t3 = rows_v[b][e, pl.ds(3 * H + j * L, L)]
                        msg_v[e, pl.ds(j * L, L)] = (
                            b0 * t0 + b1 * t1 + b2 * t2 + t3)
                return gcarry
            lax.fori_loop(0, C // L, grp_body, 0)

        def pipeline(cbase, chunks):
            pltpu.sync_copy(ed_h.at[cbase], ed_v[0])
            pltpu.async_copy(table_h.at[ed_v[0].at[0]], rows_v[0], g_sem[0])
            pltpu.async_copy(ed_h.at[cbase + 1], ed_v[1], ed_sem[1])

            def outer(i0, carry):
                for b in range(2):
                    i = i0 + b
                    nb = 1 - b
                    pltpu.make_async_copy(table_h.at[ed_v[b].at[0]],
                                          rows_v[b], g_sem[b]).wait()

                    @pl.when(i + 1 < chunks)
                    def _():
                        pltpu.make_async_copy(ed_h.at[cbase + i + 1],
                                              ed_v[nb], ed_sem[nb]).wait()
                        pltpu.async_copy(table_h.at[ed_v[nb].at[0]],
                                         rows_v[nb], g_sem[nb])

                    combine(b)
                    pltpu.sync_copy(msg_v, agg_sh.at[ed_v[b].at[1]], add=True)

                    @pl.when(i + 2 < chunks)
                    def _():
                        pltpu.async_copy(ed_h.at[cbase + i + 2],
                                         ed_v[b], ed_sem[b])
                return carry
            lax.fori_loop(0, chunks // 2, lambda t, cc: outer(t * 2, cc), 0,
                          unroll=1)

        @pl.when(c == 0)
        def _():
            pipeline(s * k0, k0)

        @pl.when(c == 1)
        def _():
            pipeline(NS * k0 + s * k1, k1)

        plsc.subcore_barrier()
        pltpu.sync_copy(agg_sh.at[pl.ds(s * rows_per_tile, rows_per_tile)],
                        out_h.at[c, pl.ds(s * rows_per_tile, rows_per_tile)])

    return k(table, ed)


def kernel(x, edge_index, edge_attr, nn1_W, nn1_b, root1, bias1,
           nn2_W, nn2_b, root2, bias2, fc1_W, fc1_b, fc2_W, fc2_b):
    N, DIM = x.shape
    E = edge_index.shape[1]
    H = root1.shape[1]

    e_pad = -(-E // (NW * C)) * (NW * C)
    pad = e_pad - E
    src_p = jnp.concatenate([edge_index[0],
                             jnp.zeros((pad,), jnp.int32)])
    dst_p = jnp.concatenate([edge_index[1],
                             jnp.full((pad,), N, jnp.int32)])
    ap = jnp.concatenate([edge_attr, jnp.zeros((pad, DIM), jnp.float32)],
                         axis=0)
    abits = lax.bitcast_convert_type(ap, jnp.int32)
    ed = (jnp.stack([src_p, dst_p, abits[:, 0], abits[:, 1], abits[:, 2]],
                    axis=0)
          .reshape(5, e_pad // C, C)
          .transpose(1, 0, 2))

    aggr = -(-(N + 1) // (NS * C)) * (NS * C)

    w1 = nn1_W.reshape(DIM, DIM, H)
    wfull1 = jnp.concatenate([w1[0], w1[1], w1[2],
                              nn1_b.reshape(DIM, H)], axis=1)
    w2 = nn2_W.reshape(DIM, H, H)
    wfull2 = jnp.concatenate([w2[0], w2[1], w2[2],
                              nn2_b.reshape(H, H)], axis=1)

    bias1_2d = bias1.reshape(1, H)
    bias2_2d = bias2.reshape(1, H)
    fc1_b2d = fc1_b.reshape(1, H)
    fc2_b2d = fc2_b.reshape(1, 1)

    t1 = _dense_table(x, wfull1)
    parts1 = _edge_pass(t1, ed, aggr, e_pad)
    p10 = parts1[0, :N, :]
    p11 = parts1[1, :N, :]
    h1, t2 = _combine_relu_table(p10, p11, x, root1, bias1_2d, wfull2)

    parts2 = _edge_pass(t2, ed, aggr, e_pad)
    p20 = parts2[0, :N, :]
    p21 = parts2[1, :N, :]

    return _final_mlp(p20, p21, h1, root2, bias2_2d, fc1_W, fc1_b2d,
                      fc2_W, fc2_b2d)


# speedup vs baseline: 1.1964x; 1.1964x over previous
import functools

import jax
import jax.numpy as jnp
from jax import lax
from jax.experimental import pallas as pl
from jax.experimental.pallas import tpu as pltpu
from jax.experimental.pallas import tpu_sc as plsc

NC = 2
NS = 16
L = 16
NW = NC * NS
C = 128
K0_NUM, K1_NUM, K_DEN = 52, 28, 80


def _dense_table(h, wfull):
    def body(h_ref, w_ref, o_ref):
        o_ref[...] = jnp.dot(h_ref[...], w_ref[...],
                             preferred_element_type=jnp.float32,
                             precision=jax.lax.Precision.HIGHEST)
    return pl.pallas_call(
        body,
        out_shape=jax.ShapeDtypeStruct((h.shape[0], wfull.shape[1]),
                                       jnp.float32),
    )(h, wfull)


def _combine_relu_table(p0, p1, h, root, bias2d, wfull):
    N = h.shape[0]
    H = root.shape[1]
    def body(p0_ref, p1_ref, h_ref, r_ref, b_ref, w_ref, ho_ref, t_ref):
        hn = p0_ref[...] + p1_ref[...] + jnp.dot(
            h_ref[...], r_ref[...], preferred_element_type=jnp.float32,
                             precision=jax.lax.Precision.HIGHEST)
        hn = jnp.maximum(hn + b_ref[...], 0.0)
        ho_ref[...] = hn
        t_ref[...] = jnp.dot(hn, w_ref[...],
                             preferred_element_type=jnp.float32,
                             precision=jax.lax.Precision.HIGHEST)
    return pl.pallas_call(
        body,
        out_shape=[jax.ShapeDtypeStruct((N, H), jnp.float32),
                   jax.ShapeDtypeStruct((N, wfull.shape[1]), jnp.float32)],
    )(p0, p1, h, root, bias2d, wfull)


def _final_mlp(p0, p1, h, root, bias2d, fc1_W, fc1_b2d, fc2_W, fc2_b2d):
    N = h.shape[0]
    def body(p0_ref, p1_ref, h_ref, r_ref, b_ref, w1_ref, b1_ref,
             w2_ref, b2_ref, o_ref):
        hn = p0_ref[...] + p1_ref[...] + jnp.dot(
            h_ref[...], r_ref[...], preferred_element_type=jnp.float32,
                             precision=jax.lax.Precision.HIGHEST)
        hn = jnp.maximum(hn + b_ref[...], 0.0)
        hn = jnp.maximum(
            jnp.dot(hn, w1_ref[...], preferred_element_type=jnp.float32,
                             precision=jax.lax.Precision.HIGHEST)
            + b1_ref[...], 0.0)
        o_ref[...] = jnp.dot(hn, w2_ref[...],
                             preferred_element_type=jnp.float32,
                             precision=jax.lax.Precision.HIGHEST) + b2_ref[...]
    return pl.pallas_call(
        body,
        out_shape=jax.ShapeDtypeStruct((N, fc2_W.shape[1]), jnp.float32),
    )(p0, p1, h, root, bias2d, fc1_W, fc1_b2d, fc2_W, fc2_b2d)


def _edge_pass(table, ed, aggr, e_pad):
    H = table.shape[1] // 4
    per_pair = e_pad // (NS * C)
    if per_pair >= 8:
        k0 = max(2, (per_pair * K0_NUM // K_DEN) & ~1)
    else:
        k0 = (per_pair // 2) & ~1
    k1 = per_pair - k0
    rows_per_tile = aggr // NS

    mesh = plsc.VectorSubcoreMesh(core_axis_name="c", subcore_axis_name="s",
                                  num_cores=NC, num_subcores=NS)

    @functools.partial(
        pl.kernel,
        out_type=jax.ShapeDtypeStruct((NC, aggr, H), jnp.float32),
        mesh=mesh,
        compiler_params=pltpu.CompilerParams(use_tc_tiling_on_sc=False,
                                             needs_layout_passes=False),
        scratch_types=[
            [pltpu.VMEM((5, C), jnp.int32)] * 2,
            [pltpu.VMEM((C, 4 * H), jnp.float32)] * 2,
            pltpu.VMEM((C, H), jnp.float32),
            pltpu.VMEM_SHARED((aggr, H), jnp.float32),
            [pltpu.SemaphoreType.DMA] * 2,
            [pltpu.SemaphoreType.DMA] * 2,
        ],
    )
    def k(table_h, ed_h, out_h, ed_v, rows_v, msg_v, agg_sh, ed_sem, g_sem):
        c = lax.axis_index("c")
        s = lax.axis_index("s")

        def zbody(e, carry):
            msg_v[e, pl.ds(0, L)] = jnp.zeros((L,), jnp.float32)
            msg_v[e, pl.ds(L, L)] = jnp.zeros((L,), jnp.float32)
            return carry
        lax.fori_loop(0, C, zbody, 0)
        for kb in range(rows_per_tile // C):
            pltpu.sync_copy(msg_v,
                            agg_sh.at[pl.ds(s * rows_per_tile + kb * C, C)])
        plsc.subcore_barrier()

        def combine(b):
            def grp_body(g, gcarry):
                av0 = plsc.bitcast(ed_v[b][2, pl.ds(g * L, L)], jnp.float32)
                av1 = plsc.bitcast(ed_v[b][3, pl.ds(g * L, L)], jnp.float32)
                av2 = plsc.bitcast(ed_v[b][4, pl.ds(g * L, L)], jnp.float32)
                for lane in range(L):
                    e = g * L + lane
                    b0 = av0[lane]
                    b1 = av1[lane]
                    b2 = av2[lane]
                    for j in range(H // L):
                        t0 = rows_v[b][e, pl.ds(j * L, L)]
                        t1 = rows_v[b][e, pl.ds(H + j * L, L)]
                        t2 = rows_v[b][e, pl.ds(2 * H + j * L, L)]
                        t3 = rows_v[b][e, pl.ds(3 * H + j * L, L)]
                        msg_v[e, pl.ds(j * L, L)] = (
                            b0 * t0 + b1 * t1 + b2 * t2 + t3)
                return gcarry
            lax.fori_loop(0, C // L, grp_body, 0)

        def pipeline(cbase, chunks):
            pltpu.sync_copy(ed_h.at[cbase], ed_v[0])
            pltpu.async_copy(table_h.at[ed_v[0].at[0]], rows_v[0], g_sem[0])
            pltpu.async_copy(ed_h.at[cbase + 1], ed_v[1], ed_sem[1])

            def outer(i0, carry):
                for b in range(2):
                    i = i0 + b
                    nb = 1 - b
                    pltpu.make_async_copy(table_h.at[ed_v[b].at[0]],
                                          rows_v[b], g_sem[b]).wait()

                    @pl.when(i + 1 < chunks)
                    def _():
                        pltpu.make_async_copy(ed_h.at[cbase + i + 1],
                                              ed_v[nb], ed_sem[nb]).wait()
                        pltpu.async_copy(table_h.at[ed_v[nb].at[0]],
                                         rows_v[nb], g_sem[nb])

                    combine(b)
                    pltpu.sync_copy(msg_v, agg_sh.at[ed_v[b].at[1]], add=True)

                    @pl.when(i + 2 < chunks)
                    def _():
                        pltpu.async_copy(ed_h.at[cbase + i + 2],
                                         ed_v[b], ed_sem[b])
                return carry
            lax.fori_loop(0, chunks // 2, lambda t, cc: outer(t * 2, cc), 0,
                          unroll=1)

        @pl.when(c == 0)
        def _():
            pipeline(s * k0, k0)

        @pl.when(c == 1)
        def _():
            pipeline(NS * k0 + s * k1, k1)

        plsc.subcore_barrier()
        pltpu.sync_copy(agg_sh.at[pl.ds(s * rows_per_tile, rows_per_tile)],
                        out_h.at[c, pl.ds(s * rows_per_tile, rows_per_tile)])

    return k(table, ed)


def kernel(x, edge_index, edge_attr, nn1_W, nn1_b, root1, bias1,
           nn2_W, nn2_b, root2, bias2, fc1_W, fc1_b, fc2_W, fc2_b):
    N, DIM = x.shape
    E = edge_index.shape[1]
    H = root1.shape[1]

    e_pad = -(-E // (NW * C)) * (NW * C)
    pad = e_pad - E
    src_p = jnp.concatenate([edge_index[0],
                             jnp.zeros((pad,), jnp.int32)])
    dst_p = jnp.concatenate([edge_index[1],
                             jnp.full((pad,), N, jnp.int32)])
    ap = jnp.concatenate([edge_attr, jnp.zeros((pad, DIM), jnp.float32)],
                         axis=0)
    abits = lax.bitcast_convert_type(ap, jnp.int32)
    ed = (jnp.stack([src_p, dst_p, abits[:, 0], abits[:, 1], abits[:, 2]],
                    axis=0)
          .reshape(5, e_pad // C, C)
          .transpose(1, 0, 2))

    aggr = -(-(N + 1) // (NS * C)) * (NS * C)

    w1 = nn1_W.reshape(DIM, DIM, H)
    wfull1 = jnp.concatenate([w1[0], w1[1], w1[2],
                              nn1_b.reshape(DIM, H)], axis=1)
    w2 = nn2_W.reshape(DIM, H, H)
    wfull2 = jnp.concatenate([w2[0], w2[1], w2[2],
                              nn2_b.reshape(H, H)], axis=1)

    bias1_2d = bias1.reshape(1, H)
    bias2_2d = bias2.reshape(1, H)
    fc1_b2d = fc1_b.reshape(1, H)
    fc2_b2d = fc2_b.reshape(1, 1)

    t1 = _dense_table(x, wfull1)
    parts1 = _edge_pass(t1, ed, aggr, e_pad)
    p10 = parts1[0, :N, :]
    p11 = parts1[1, :N, :]
    h1, t2 = _combine_relu_table(p10, p11, x, root1, bias1_2d, wfull2)

    parts2 = _edge_pass(t2, ed, aggr, e_pad)
    p20 = parts2[0, :N, :]
    p21 = parts2[1, :N, :]

    return _final_mlp(p20, p21, h1, root2, bias2_2d, fc1_W, fc1_b2d,
                      fc2_W, fc2_b2d)


# speedup vs baseline: 1.3886x; 1.1607x over previous
import functools

import jax
import jax.numpy as jnp
from jax import lax
from jax.experimental import pallas as pl
from jax.experimental.pallas import tpu as pltpu
from jax.experimental.pallas import tpu_sc as plsc

NC = 2
NS = 16
L = 16
NW = NC * NS
C = 128
K0_NUM, K1_NUM, K_DEN = 52, 28, 80


def _combine_relu_table(z0, z1, wm, h, root, bias2d, wfull):
    N = h.shape[0]
    H = root.shape[1]
    def body(z0_ref, z1_ref, wm_ref, h_ref, r_ref, b_ref, w_ref,
             ho_ref, t_ref):
        hn = jnp.dot(z0_ref[...] + z1_ref[...], wm_ref[...],
                     preferred_element_type=jnp.float32,
                     precision=jax.lax.Precision.HIGHEST) + jnp.dot(
            h_ref[...], r_ref[...], preferred_element_type=jnp.float32,
                             precision=jax.lax.Precision.HIGHEST)
        hn = jnp.maximum(hn + b_ref[...], 0.0)
        ho_ref[...] = hn
        t_ref[...] = jnp.dot(hn, w_ref[...],
                             preferred_element_type=jnp.float32,
                             precision=jax.lax.Precision.HIGHEST)
    return pl.pallas_call(
        body,
        out_shape=[jax.ShapeDtypeStruct((N, H), jnp.float32),
                   jax.ShapeDtypeStruct((N, wfull.shape[1]), jnp.float32)],
    )(z0, z1, wm, h, root, bias2d, wfull)


def _final_mlp(p0, p1, h, root, bias2d, fc1_W, fc1_b2d, fc2_W, fc2_b2d):
    N = h.shape[0]
    def body(p0_ref, p1_ref, h_ref, r_ref, b_ref, w1_ref, b1_ref,
             w2_ref, b2_ref, o_ref):
        hn = p0_ref[...] + p1_ref[...] + jnp.dot(
            h_ref[...], r_ref[...], preferred_element_type=jnp.float32,
                             precision=jax.lax.Precision.HIGHEST)
        hn = jnp.maximum(hn + b_ref[...], 0.0)
        hn = jnp.maximum(
            jnp.dot(hn, w1_ref[...], preferred_element_type=jnp.float32,
                             precision=jax.lax.Precision.HIGHEST)
            + b1_ref[...], 0.0)
        o_ref[...] = jnp.dot(hn, w2_ref[...],
                             preferred_element_type=jnp.float32,
                             precision=jax.lax.Precision.HIGHEST) + b2_ref[...]
    return pl.pallas_call(
        body,
        out_shape=jax.ShapeDtypeStruct((N, fc2_W.shape[1]), jnp.float32),
    )(p0, p1, h, root, bias2d, fc1_W, fc1_b2d, fc2_W, fc2_b2d)


def _edge_pass(table, ed, aggr, e_pad):
    H = table.shape[1] // 4
    per_pair = e_pad // (NS * C)
    if per_pair >= 8:
        k0 = max(2, (per_pair * K0_NUM // K_DEN) & ~1)
    else:
        k0 = (per_pair // 2) & ~1
    k1 = per_pair - k0
    rows_per_tile = aggr // NS

    mesh = plsc.VectorSubcoreMesh(core_axis_name="c", subcore_axis_name="s",
                                  num_cores=NC, num_subcores=NS)

    @functools.partial(
        pl.kernel,
        out_type=jax.ShapeDtypeStruct((NC, aggr, H), jnp.float32),
        mesh=mesh,
        compiler_params=pltpu.CompilerParams(use_tc_tiling_on_sc=False,
                                             needs_layout_passes=False),
        scratch_types=[
            [pltpu.VMEM((5, C), jnp.int32)] * 2,
            [pltpu.VMEM((C, 4 * H), jnp.float32)] * 2,
            pltpu.VMEM((C, H), jnp.float32),
            pltpu.VMEM_SHARED((aggr, H), jnp.float32),
            [pltpu.SemaphoreType.DMA] * 2,
            [pltpu.SemaphoreType.DMA] * 2,
        ],
    )
    def k(table_h, ed_h, out_h, ed_v, rows_v, msg_v, agg_sh, ed_sem, g_sem):
        c = lax.axis_index("c")
        s = lax.axis_index("s")

        def zbody(e, carry):
            msg_v[e, pl.ds(0, L)] = jnp.zeros((L,), jnp.float32)
            msg_v[e, pl.ds(L, L)] = jnp.zeros((L,), jnp.float32)
            return carry
        lax.fori_loop(0, C, zbody, 0)
        for kb in range(rows_per_tile // C):
            pltpu.sync_copy(msg_v,
                            agg_sh.at[pl.ds(s * rows_per_tile + kb * C, C)])
        plsc.subcore_barrier()

        def combine(b):
            def grp_body(g, gcarry):
                av0 = plsc.bitcast(ed_v[b][2, pl.ds(g * L, L)], jnp.float32)
                av1 = plsc.bitcast(ed_v[b][3, pl.ds(g * L, L)], jnp.float32)
                av2 = plsc.bitcast(ed_v[b][4, pl.ds(g * L, L)], jnp.float32)
                for lane in range(L):
                    e = g * L + lane
                    b0 = av0[lane]
                    b1 = av1[lane]
                    b2 = av2[lane]
                    for j in range(H // L):
                        t0 = rows_v[b][e, pl.ds(j * L, L)]
                        t1 = rows_v[b][e, pl.ds(H + j * L, L)]
                        t2 = rows_v[b][e, pl.ds(2 * H + j * L, L)]
                        t3 = rows_v[b][e, pl.ds(3 * H + j * L, L)]
                        msg_v[e, pl.ds(j * L, L)] = (
                            b0 * t0 + b1 * t1 + b2 * t2 + t3)
                return gcarry
            lax.fori_loop(0, C // L, grp_body, 0)

        def pipeline(cbase, chunks):
            pltpu.sync_copy(ed_h.at[cbase], ed_v[0])
            pltpu.async_copy(table_h.at[ed_v[0].at[0]], rows_v[0], g_sem[0])
            pltpu.async_copy(ed_h.at[cbase + 1], ed_v[1], ed_sem[1])

            def outer(i0, carry):
                for b in range(2):
                    i = i0 + b
                    nb = 1 - b
                    pltpu.make_async_copy(table_h.at[ed_v[b].at[0]],
                                          rows_v[b], g_sem[b]).wait()

                    @pl.when(i + 1 < chunks)
                    def _():
                        pltpu.make_async_copy(ed_h.at[cbase + i + 1],
                                              ed_v[nb], ed_sem[nb]).wait()
                        pltpu.async_copy(table_h.at[ed_v[nb].at[0]],
                                         rows_v[nb], g_sem[nb])

                    combine(b)
                    pltpu.sync_copy(msg_v, agg_sh.at[ed_v[b].at[1]], add=True)

                    @pl.when(i + 2 < chunks)
                    def _():
                        pltpu.async_copy(ed_h.at[cbase + i + 2],
                                         ed_v[b], ed_sem[b])
                return carry
            lax.fori_loop(0, chunks // 2, lambda t, cc: outer(t * 2, cc), 0,
                          unroll=1)

        @pl.when(c == 0)
        def _():
            pipeline(s * k0, k0)

        @pl.when(c == 1)
        def _():
            pipeline(NS * k0 + s * k1, k1)

        plsc.subcore_barrier()
        pltpu.sync_copy(agg_sh.at[pl.ds(s * rows_per_tile, rows_per_tile)],
                        out_h.at[c, pl.ds(s * rows_per_tile, rows_per_tile)])

    return k(table, ed)


def _edge_pass_l1(xpad, ed1, aa1, masks, aggr, e_pad):
    per_pair = e_pad // (NS * C)
    if per_pair >= 8:
        k0 = max(2, (per_pair * K0_NUM // K_DEN) & ~1)
    else:
        k0 = (per_pair // 2) & ~1
    k1 = per_pair - k0
    rows_per_tile = aggr // NS

    mesh = plsc.VectorSubcoreMesh(core_axis_name="c", subcore_axis_name="s",
                                  num_cores=NC, num_subcores=NS)

    @functools.partial(
        pl.kernel,
        out_type=jax.ShapeDtypeStruct((NC, aggr, L), jnp.float32),
        mesh=mesh,
        compiler_params=pltpu.CompilerParams(use_tc_tiling_on_sc=False,
                                             needs_layout_passes=False),
        scratch_types=[
            [pltpu.VMEM((2, C), jnp.int32)] * 2,
            [pltpu.VMEM((C, L), jnp.float32)] * 2,
            [pltpu.VMEM((C, L), jnp.float32)] * 2,
            pltpu.VMEM((3, L), jnp.float32),
            pltpu.VMEM((C, L), jnp.float32),
            pltpu.VMEM_SHARED((aggr, L), jnp.float32),
            [pltpu.SemaphoreType.DMA] * 2,
            [pltpu.SemaphoreType.DMA] * 2,
            [pltpu.SemaphoreType.DMA] * 2,
        ],
    )
    def k(x_h, ed_h, aa_h, m_h, out_h, ed_v, aa_v, xg_v, m_v, msg_v, agg_sh,
          ed_sem, aa_sem, g_sem):
        c = lax.axis_index("c")
        s = lax.axis_index("s")

        pltpu.sync_copy(m_h, m_v)
        m0 = m_v[0, pl.ds(0, L)]
        m1 = m_v[1, pl.ds(0, L)]
        m2 = m_v[2, pl.ds(0, L)]

        def zbody(e, carry):
            msg_v[e, pl.ds(0, L)] = jnp.zeros((L,), jnp.float32)
            return carry
        lax.fori_loop(0, C, zbody, 0)
        for kb in range(rows_per_tile // C):
            pltpu.sync_copy(msg_v,
                            agg_sh.at[pl.ds(s * rows_per_tile + kb * C, C)])
        plsc.subcore_barrier()

        def combine(b):
            def grp_body(g, gcarry):
                for lane in range(L):
                    e = g * L + lane
                    xv = xg_v[b][e, pl.ds(0, L)]
                    aav = aa_v[b][e, pl.ds(0, L)]
                    msg_v[e, pl.ds(0, L)] = (
                        xv[0] * (aav * m0) + xv[1] * (aav * m1)
                        + xv[2] * (aav * m2))
                return gcarry
            lax.fori_loop(0, C // L, grp_body, 0)

        def pipeline(cbase, chunks):
            pltpu.sync_copy(ed_h.at[cbase], ed_v[0])
            pltpu.async_copy(x_h.at[ed_v[0].at[0]], xg_v[0], g_sem[0])
            pltpu.async_copy(aa_h.at[cbase], aa_v[0], aa_sem[0])
            pltpu.async_copy(ed_h.at[cbase + 1], ed_v[1], ed_sem[1])
            pltpu.async_copy(aa_h.at[cbase + 1], aa_v[1], aa_sem[1])

            def outer(i0, carry):
                for b in range(2):
                    i = i0 + b
                    nb = 1 - b
                    pltpu.make_async_copy(x_h.at[ed_v[b].at[0]],
                                          xg_v[b], g_sem[b]).wait()

                    @pl.when(i + 1 < chunks)
                    def _():
                        pltpu.make_async_copy(ed_h.at[cbase + i + 1],
                                              ed_v[nb], ed_sem[nb]).wait()
                        pltpu.async_copy(x_h.at[ed_v[nb].at[0]],
                                         xg_v[nb], g_sem[nb])

                    pltpu.make_async_copy(aa_h.at[cbase + i],
                                          aa_v[b], aa_sem[b]).wait()
                    combine(b)
                    pltpu.sync_copy(msg_v, agg_sh.at[ed_v[b].at[1]], add=True)

                    @pl.when(i + 2 < chunks)
                    def _():
                        pltpu.async_copy(ed_h.at[cbase + i + 2],
                                         ed_v[b], ed_sem[b])
                        pltpu.async_copy(aa_h.at[cbase + i + 2],
                                         aa_v[b], aa_sem[b])
                return carry
            lax.fori_loop(0, chunks // 2, lambda t, cc: outer(t * 2, cc), 0,
                          unroll=1)

        @pl.when(c == 0)
        def _():
            pipeline(s * k0, k0)

        @pl.when(c == 1)
        def _():
            pipeline(NS * k0 + s * k1, k1)

        plsc.subcore_barrier()
        pltpu.sync_copy(agg_sh.at[pl.ds(s * rows_per_tile, rows_per_tile)],
                        out_h.at[c, pl.ds(s * rows_per_tile, rows_per_tile)])

    return k(xpad, ed1, aa1, masks)


def kernel(x, edge_index, edge_attr, nn1_W, nn1_b, root1, bias1,
           nn2_W, nn2_b, root2, bias2, fc1_W, fc1_b, fc2_W, fc2_b):
    N, DIM = x.shape
    E = edge_index.shape[1]
    H = root1.shape[1]

    e_pad = -(-E // (NW * C)) * (NW * C)
    pad = e_pad - E
    src_p = jnp.concatenate([edge_index[0],
                             jnp.zeros((pad,), jnp.int32)])
    dst_p = jnp.concatenate([edge_index[1],
                             jnp.full((pad,), N, jnp.int32)])
    ap = jnp.concatenate([edge_attr, jnp.zeros((pad, DIM), jnp.float32)],
                         axis=0)
    abits = lax.bitcast_convert_type(ap, jnp.int32)
    ed = (jnp.stack([src_p, dst_p, abits[:, 0], abits[:, 1], abits[:, 2]],
                    axis=0)
          .reshape(5, e_pad // C, C)
          .transpose(1, 0, 2))

    aggr = -(-(N + 1) // (NS * C)) * (NS * C)

    xpad = jnp.concatenate([x, jnp.zeros((N, 16 - DIM), jnp.float32)],
                           axis=1)
    ed1 = (jnp.stack([src_p, dst_p], axis=0)
           .reshape(2, e_pad // C, C)
           .transpose(1, 0, 2))
    a4 = jnp.concatenate([ap, jnp.ones((e_pad, 1), jnp.float32)], axis=1)
    aa1 = jnp.tile(a4, (1, 4)).reshape(e_pad // C, C, 16)
    masks = jnp.asarray(
        [[1.0 if lane // 4 == i else 0.0 for lane in range(16)]
         for i in range(DIM)], jnp.float32)
    w1 = nn1_W.reshape(DIM, DIM, H)
    b3 = nn1_b.reshape(DIM, H)
    wm_rows = [w1[d][i] if d < DIM else b3[i]
               for i in range(DIM) for d in range(DIM + 1)]
    wm_rows += [jnp.zeros((H,), jnp.float32)] * (16 - len(wm_rows))
    wmat16 = jnp.stack(wm_rows, axis=0)

    w2 = nn2_W.reshape(DIM, H, H)
    wfull2 = jnp.concatenate([w2[0], w2[1], w2[2],
                              nn2_b.reshape(H, H)], axis=1)

    bias1_2d = bias1.reshape(1, H)
    bias2_2d = bias2.reshape(1, H)
    fc1_b2d = fc1_b.reshape(1, H)
    fc2_b2d = fc2_b.reshape(1, 1)

    parts1 = _edge_pass_l1(xpad, ed1, aa1, masks, aggr, e_pad)
    z10 = parts1[0, :N, :]
    z11 = parts1[1, :N, :]
    h1, t2 = _combine_relu_table(z10, z11, wmat16, x, root1, bias1_2d,
                                 wfull2)

    parts2 = _edge_pass(t2, ed, aggr, e_pad)
    p20 = parts2[0, :N, :]
    p21 = parts2[1, :N, :]

    return _final_mlp(p20, p21, h1, root2, bias2_2d, fc1_W, fc1_b2d,
                      fc2_W, fc2_b2d)


# speedup vs baseline: 1.3981x; 1.0069x over previous
import functools

import jax
import jax.numpy as jnp
from jax import lax
from jax.experimental import pallas as pl
from jax.experimental.pallas import tpu as pltpu
from jax.experimental.pallas import tpu_sc as plsc

NC = 2
NS = 16
L = 16
NW = NC * NS
C = 128
K0_NUM, K1_NUM, K_DEN = 52, 28, 80


def _combine_relu_table(z0, z1, wm, h, root, bias2d, wfull):
    N = h.shape[0]
    H = root.shape[1]
    def body(z0_ref, z1_ref, wm_ref, h_ref, r_ref, b_ref, w_ref,
             ho_ref, t_ref):
        hn = jnp.dot(z0_ref[...] + z1_ref[...], wm_ref[...],
                     preferred_element_type=jnp.float32,
                     precision=jax.lax.Precision.HIGHEST) + jnp.dot(
            h_ref[...], r_ref[...], preferred_element_type=jnp.float32,
                             precision=jax.lax.Precision.HIGHEST)
        hn = jnp.maximum(hn + b_ref[...], 0.0)
        ho_ref[...] = hn
        t_ref[...] = jnp.dot(hn, w_ref[...],
                             preferred_element_type=jnp.float32,
                             precision=jax.lax.Precision.HIGHEST)
    return pl.pallas_call(
        body,
        out_shape=[jax.ShapeDtypeStruct((N, H), jnp.float32),
                   jax.ShapeDtypeStruct((N, wfull.shape[1]), jnp.float32)],
    )(z0, z1, wm, h, root, bias2d, wfull)


def _final_mlp(p0, p1, h, root, bias2d, fc1_W, fc1_b2d, fc2_W, fc2_b2d):
    N = h.shape[0]
    def body(p0_ref, p1_ref, h_ref, r_ref, b_ref, w1_ref, b1_ref,
             w2_ref, b2_ref, o_ref):
        hn = p0_ref[...] + p1_ref[...] + jnp.dot(
            h_ref[...], r_ref[...], preferred_element_type=jnp.float32,
                             precision=jax.lax.Precision.HIGHEST)
        hn = jnp.maximum(hn + b_ref[...], 0.0)
        hn = jnp.maximum(
            jnp.dot(hn, w1_ref[...], preferred_element_type=jnp.float32,
                             precision=jax.lax.Precision.HIGHEST)
            + b1_ref[...], 0.0)
        o_ref[...] = jnp.dot(hn, w2_ref[...],
                             preferred_element_type=jnp.float32,
                             precision=jax.lax.Precision.HIGHEST) + b2_ref[...]
    return pl.pallas_call(
        body,
        out_shape=jax.ShapeDtypeStruct((N, fc2_W.shape[1]), jnp.float32),
    )(p0, p1, h, root, bias2d, fc1_W, fc1_b2d, fc2_W, fc2_b2d)


def _edge_pass(table, ed, aggr, e_pad, k0_num=K0_NUM):
    H = table.shape[1] // 4
    per_pair = e_pad // (NS * C)
    if per_pair >= 8:
        k0 = max(2, (per_pair * k0_num // K_DEN) & ~1)
    else:
        k0 = (per_pair // 2) & ~1
    k1 = per_pair - k0
    rows_per_tile = aggr // NS

    mesh = plsc.VectorSubcoreMesh(core_axis_name="c", subcore_axis_name="s",
                                  num_cores=NC, num_subcores=NS)

    @functools.partial(
        pl.kernel,
        out_type=jax.ShapeDtypeStruct((NC, aggr, H), jnp.float32),
        mesh=mesh,
        compiler_params=pltpu.CompilerParams(use_tc_tiling_on_sc=False,
                                             needs_layout_passes=False),
        scratch_types=[
            [pltpu.VMEM((5, C), jnp.int32)] * 2,
            [pltpu.VMEM((C, 4 * H), jnp.float32)] * 2,
            pltpu.VMEM((C, H), jnp.float32),
            pltpu.VMEM_SHARED((aggr, H), jnp.float32),
            [pltpu.SemaphoreType.DMA] * 2,
            [pltpu.SemaphoreType.DMA] * 2,
        ],
    )
    def k(table_h, ed_h, out_h, ed_v, rows_v, msg_v, agg_sh, ed_sem, g_sem):
        c = lax.axis_index("c")
        s = lax.axis_index("s")

        def zbody(e, carry):
            msg_v[e, pl.ds(0, L)] = jnp.zeros((L,), jnp.float32)
            msg_v[e, pl.ds(L, L)] = jnp.zeros((L,), jnp.float32)
            return carry
        lax.fori_loop(0, C, zbody, 0)
        for kb in range(rows_per_tile // C):
            pltpu.sync_copy(msg_v,
                            agg_sh.at[pl.ds(s * rows_per_tile + kb * C, C)])
        plsc.subcore_barrier()

        def combine(b):
            def grp_body(g, gcarry):
                av0 = plsc.bitcast(ed_v[b][2, pl.ds(g * L, L)], jnp.float32)
                av1 = plsc.bitcast(ed_v[b][3, pl.ds(g * L, L)], jnp.float32)
                av2 = plsc.bitcast(ed_v[b][4, pl.ds(g * L, L)], jnp.float32)
                for lane in range(L):
                    e = g * L + lane
                    b0 = av0[lane]
                    b1 = av1[lane]
                    b2 = av2[lane]
                    for j in range(H // L):
                        t0 = rows_v[b][e, pl.ds(j * L, L)]
                        t1 = rows_v[b][e, pl.ds(H + j * L, L)]
                        t2 = rows_v[b][e, pl.ds(2 * H + j * L, L)]
                        t3 = rows_v[b][e, pl.ds(3 * H + j * L, L)]
                        msg_v[e, pl.ds(j * L, L)] = (
                            b0 * t0 + b1 * t1 + b2 * t2 + t3)
                return gcarry
            lax.fori_loop(0, C // L, grp_body, 0)

        def pipeline(cbase, chunks):
            pltpu.sync_copy(ed_h.at[cbase], ed_v[0])
            pltpu.async_copy(table_h.at[ed_v[0].at[0]], rows_v[0], g_sem[0])
            pltpu.async_copy(ed_h.at[cbase + 1], ed_v[1], ed_sem[1])

            def outer(i0, carry):
                for b in range(2):
                    i = i0 + b
                    nb = 1 - b
                    pltpu.make_async_copy(table_h.at[ed_v[b].at[0]],
                                          rows_v[b], g_sem[b]).wait()

                    @pl.when(i + 1 < chunks)
                    def _():
                        pltpu.make_async_copy(ed_h.at[cbase + i + 1],
                                              ed_v[nb], ed_sem[nb]).wait()
                        pltpu.async_copy(table_h.at[ed_v[nb].at[0]],
                                         rows_v[nb], g_sem[nb])

                    combine(b)
                    pltpu.sync_copy(msg_v, agg_sh.at[ed_v[b].at[1]], add=True)

                    @pl.when(i + 2 < chunks)
                    def _():
                        pltpu.async_copy(ed_h.at[cbase + i + 2],
                                         ed_v[b], ed_sem[b])
                return carry
            lax.fori_loop(0, chunks // 2, lambda t, cc: outer(t * 2, cc), 0,
                          unroll=1)

        @pl.when(c == 0)
        def _():
            pipeline(s * k0, k0)

        @pl.when(c == 1)
        def _():
            pipeline(NS * k0 + s * k1, k1)

        plsc.subcore_barrier()
        pltpu.sync_copy(agg_sh.at[pl.ds(s * rows_per_tile, rows_per_tile)],
                        out_h.at[c, pl.ds(s * rows_per_tile, rows_per_tile)])

    return k(table, ed)


def _edge_pass_l1(xpad, ed1, aa1, masks, aggr, e_pad):
    per_pair = e_pad // (NS * C)
    if per_pair >= 8:
        k0 = max(2, (per_pair * K0_NUM // K_DEN) & ~1)
    else:
        k0 = (per_pair // 2) & ~1
    k1 = per_pair - k0
    rows_per_tile = aggr // NS

    mesh = plsc.VectorSubcoreMesh(core_axis_name="c", subcore_axis_name="s",
                                  num_cores=NC, num_subcores=NS)

    @functools.partial(
        pl.kernel,
        out_type=jax.ShapeDtypeStruct((NC, aggr, L), jnp.float32),
        mesh=mesh,
        compiler_params=pltpu.CompilerParams(use_tc_tiling_on_sc=False,
                                             needs_layout_passes=False),
        scratch_types=[
            [pltpu.VMEM((2, C), jnp.int32)] * 2,
            [pltpu.VMEM((C, L), jnp.float32)] * 2,
            [pltpu.VMEM((C, L), jnp.float32)] * 2,
            pltpu.VMEM((3, L), jnp.float32),
            pltpu.VMEM((C, L), jnp.float32),
            pltpu.VMEM_SHARED((aggr, L), jnp.float32),
            [pltpu.SemaphoreType.DMA] * 2,
            [pltpu.SemaphoreType.DMA] * 2,
            [pltpu.SemaphoreType.DMA] * 2,
        ],
    )
    def k(x_h, ed_h, aa_h, m_h, out_h, ed_v, aa_v, xg_v, m_v, msg_v, agg_sh,
          ed_sem, aa_sem, g_sem):
        c = lax.axis_index("c")
        s = lax.axis_index("s")

        pltpu.sync_copy(m_h, m_v)
        m0 = m_v[0, pl.ds(0, L)]
        m1 = m_v[1, pl.ds(0, L)]
        m2 = m_v[2, pl.ds(0, L)]

        def zbody(e, carry):
            msg_v[e, pl.ds(0, L)] = jnp.zeros((L,), jnp.float32)
            return carry
        lax.fori_loop(0, C, zbody, 0)
        for kb in range(rows_per_tile // C):
            pltpu.sync_copy(msg_v,
                            agg_sh.at[pl.ds(s * rows_per_tile + kb * C, C)])
        plsc.subcore_barrier()

        def combine(b):
            def grp_body(g, gcarry):
                for lane in range(L):
                    e = g * L + lane
                    xv = xg_v[b][e, pl.ds(0, L)]
                    aav = aa_v[b][e, pl.ds(0, L)]
                    msg_v[e, pl.ds(0, L)] = (
                        xv[0] * (aav * m0) + xv[1] * (aav * m1)
                        + xv[2] * (aav * m2))
                return gcarry
            lax.fori_loop(0, C // L, grp_body, 0)

        def pipeline(cbase, chunks):
            pltpu.sync_copy(ed_h.at[cbase], ed_v[0])
            pltpu.async_copy(x_h.at[ed_v[0].at[0]], xg_v[0], g_sem[0])
            pltpu.async_copy(aa_h.at[cbase], aa_v[0], aa_sem[0])
            pltpu.async_copy(ed_h.at[cbase + 1], ed_v[1], ed_sem[1])
            pltpu.async_copy(aa_h.at[cbase + 1], aa_v[1], aa_sem[1])

            def outer(i0, carry):
                for b in range(2):
                    i = i0 + b
                    nb = 1 - b
                    pltpu.make_async_copy(x_h.at[ed_v[b].at[0]],
                                          xg_v[b], g_sem[b]).wait()

                    @pl.when(i + 1 < chunks)
                    def _():
                        pltpu.make_async_copy(ed_h.at[cbase + i + 1],
                                              ed_v[nb], ed_sem[nb]).wait()
                        pltpu.async_copy(x_h.at[ed_v[nb].at[0]],
                                         xg_v[nb], g_sem[nb])

                    pltpu.make_async_copy(aa_h.at[cbase + i],
                                          aa_v[b], aa_sem[b]).wait()
                    combine(b)
                    pltpu.sync_copy(msg_v, agg_sh.at[ed_v[b].at[1]], add=True)

                    @pl.when(i + 2 < chunks)
                    def _():
                        pltpu.async_copy(ed_h.at[cbase + i + 2],
                                         ed_v[b], ed_sem[b])
                        pltpu.async_copy(aa_h.at[cbase + i + 2],
                                         aa_v[b], aa_sem[b])
                return carry
            lax.fori_loop(0, chunks // 2, lambda t, cc: outer(t * 2, cc), 0,
                          unroll=1)

        @pl.when(c == 0)
        def _():
            pipeline(s * k0, k0)

        @pl.when(c == 1)
        def _():
            pipeline(NS * k0 + s * k1, k1)

        plsc.subcore_barrier()
        pltpu.sync_copy(agg_sh.at[pl.ds(s * rows_per_tile, rows_per_tile)],
                        out_h.at[c, pl.ds(s * rows_per_tile, rows_per_tile)])

    return k(xpad, ed1, aa1, masks)


def kernel(x, edge_index, edge_attr, nn1_W, nn1_b, root1, bias1,
           nn2_W, nn2_b, root2, bias2, fc1_W, fc1_b, fc2_W, fc2_b):
    N, DIM = x.shape
    E = edge_index.shape[1]
    H = root1.shape[1]

    e_pad = -(-E // (NW * C)) * (NW * C)
    pad = e_pad - E
    src_p = jnp.concatenate([edge_index[0],
                             jnp.zeros((pad,), jnp.int32)])
    dst_p = jnp.concatenate([edge_index[1],
                             jnp.full((pad,), N, jnp.int32)])
    ap = jnp.concatenate([edge_attr, jnp.zeros((pad, DIM), jnp.float32)],
                         axis=0)
    abits = lax.bitcast_convert_type(ap, jnp.int32)
    ed = (jnp.stack([src_p, dst_p, abits[:, 0], abits[:, 1], abits[:, 2]],
                    axis=0)
          .reshape(5, e_pad // C, C)
          .transpose(1, 0, 2))

    aggr = -(-(N + 1) // (NS * C)) * (NS * C)

    xpad = jnp.concatenate([x, jnp.zeros((N, 16 - DIM), jnp.float32)],
                           axis=1)
    ed1 = (jnp.stack([src_p, dst_p], axis=0)
           .reshape(2, e_pad // C, C)
           .transpose(1, 0, 2))
    a4 = jnp.concatenate([ap, jnp.ones((e_pad, 1), jnp.float32)], axis=1)
    aa1 = jnp.tile(a4, (1, 4)).reshape(e_pad // C, C, 16)
    masks = jnp.asarray(
        [[1.0 if lane // 4 == i else 0.0 for lane in range(16)]
         for i in range(DIM)], jnp.float32)
    w1 = nn1_W.reshape(DIM, DIM, H)
    b3 = nn1_b.reshape(DIM, H)
    wm_rows = [w1[d][i] if d < DIM else b3[i]
               for i in range(DIM) for d in range(DIM + 1)]
    wm_rows += [jnp.zeros((H,), jnp.float32)] * (16 - len(wm_rows))
    wmat16 = jnp.stack(wm_rows, axis=0)

    w2 = nn2_W.reshape(DIM, H, H)
    wfull2 = jnp.concatenate([w2[0], w2[1], w2[2],
                              nn2_b.reshape(H, H)], axis=1)

    bias1_2d = bias1.reshape(1, H)
    bias2_2d = bias2.reshape(1, H)
    fc1_b2d = fc1_b.reshape(1, H)
    fc2_b2d = fc2_b.reshape(1, 1)

    parts1 = _edge_pass_l1(xpad, ed1, aa1, masks, aggr, e_pad)
    z10 = parts1[0, :N, :]
    z11 = parts1[1, :N, :]
    h1, t2 = _combine_relu_table(z10, z11, wmat16, x, root1, bias1_2d,
                                 wfull2)

    parts2 = _edge_pass(t2, ed, aggr, e_pad, k0_num=56)
    p20 = parts2[0, :N, :]
    p21 = parts2[1, :N, :]

    return _final_mlp(p20, p21, h1, root2, bias2_2d, fc1_W, fc1_b2d,
                      fc2_W, fc2_b2d)
